# baseline (speedup 1.0000x reference)
import jax
import jax.numpy as jnp
from jax.experimental import pallas as pl

_HID = [128, 128]


def _gine_conv(x, ei, ea, p):
    src = ei[0]
    dst = ei[1]
    e = ea @ p["lin"]["w"] + p["lin"]["b"]
    msg = jax.nn.relu(x[src] + e)
    aggr = jax.ops.segment_sum(msg, dst, num_segments=x.shape[0])
    h = x + aggr
    h = jax.nn.relu(h @ p["mlp1"]["w"] + p["mlp1"]["b"])
    return h @ p["mlp2"]["w"] + p["mlp2"]["b"]


def _bn(x, g, b):
    m = jnp.mean(x, axis=0)
    v = jnp.var(x, axis=0)
    return (x - m) / jnp.sqrt(v + 1e-5) * g + b


def _head_kernel(x_ref, w1_ref, b1_ref, w2_ref, b2_ref, o_ref):
    g = jnp.mean(x_ref[...], axis=0, keepdims=True)
    h = jax.nn.relu(g @ w1_ref[...] + b1_ref[...][None, :])
    o_ref[...] = h @ w2_ref[...] + b2_ref[...][None, :]


def kernel(node_features, fwd_edges_index, bwd_edges_index, edge_attr, params):
    x = node_features[0]
    ea = edge_attr[0]
    fei = fwd_edges_index[0]
    bei = bwd_edges_index[0]
    for l in range(len(_HID)):
        lp = params["layer%d" % l]
        fx = _gine_conv(x, fei, ea, lp["fwd"])
        bx = _gine_conv(x, bei, ea, lp["bwd"])
        x = jnp.concatenate([fx, bx], axis=1)
        x = x @ lp["merge"]["w"] + lp["merge"]["b"]
        x = _bn(x, lp["bn_g"], lp["bn_b"])
        x = jax.nn.leaky_relu(x, 0.01)
    out = pl.pallas_call(
        _head_kernel,
        out_shape=jax.ShapeDtypeStruct((1, params["lin2"]["w"].shape[1]),
                                       jnp.float32),
    )(x, params["lin1"]["w"], params["lin1"]["b"],
      params["lin2"]["w"], params["lin2"]["b"])
    return out
